# baseline (speedup 1.0000x reference)
import functools

import jax
import jax.numpy as jnp
from jax import lax
from jax.experimental import pallas as pl
from jax.experimental.pallas import tpu as pltpu
from jax.experimental.pallas import tpu_sc as plsc


@functools.lru_cache(maxsize=None)
def _make_gather(V, D, B, C):
    info = plsc.get_sparse_core_info()
    NC, NS = info.num_cores, info.num_subcores
    NW = NC * NS
    assert B % NW == 0
    b_per_w = B // NW
    assert b_per_w % C == 0
    n_chunks = b_per_w // C

    mesh = plsc.VectorSubcoreMesh(core_axis_name="c", subcore_axis_name="s")

    @functools.partial(
        pl.kernel,
        mesh=mesh,
        out_type=jax.ShapeDtypeStruct((B, D), jnp.float32),
        scratch_types=[
            pltpu.VMEM((C,), jnp.int32),
            pltpu.VMEM((C, D), jnp.float32),
            pltpu.SemaphoreType.DMA,
        ],
        compiler_params=pltpu.CompilerParams(use_tc_tiling_on_sc=False),
    )
    def k(table_hbm, idx_hbm, out_hbm, idx_v, rows_v, sem):
        wid = lax.axis_index("s") * NC + lax.axis_index("c")
        base = wid * b_per_w

        def body(i, carry):
            off = base + i * C
            pltpu.sync_copy(idx_hbm.at[pl.ds(off, C)], idx_v)
            pltpu.async_copy(table_hbm.at[idx_v], rows_v, sem).wait()
            pltpu.sync_copy(rows_v, out_hbm.at[pl.ds(off, C)])
            return carry

        lax.fori_loop(0, n_chunks, body, 0)

    return k


def kernel(x, table):
    BATCH, HIST = x.shape
    V, D = table.shape
    B = BATCH * HIST
    xf = x.reshape(B).astype(jnp.int32)
    out = _make_gather(V, D, B, 800)(table, xf)
    return out.reshape(BATCH, HIST, D)


# speedup vs baseline: 1.0250x; 1.0250x over previous
import functools

import jax
import jax.numpy as jnp
from jax import lax
from jax.experimental import pallas as pl
from jax.experimental.pallas import tpu as pltpu
from jax.experimental.pallas import tpu_sc as plsc

_NBUF = 4
_LEAD = 2


@functools.lru_cache(maxsize=None)
def _make_gather(V, D, B, C):
    info = plsc.get_sparse_core_info()
    NC, NS = info.num_cores, info.num_subcores
    NW = NC * NS
    assert B % NW == 0
    b_per_w = B // NW
    assert b_per_w % (C * _NBUF) == 0
    n_chunks = b_per_w // C

    mesh = plsc.VectorSubcoreMesh(core_axis_name="c", subcore_axis_name="s")

    @functools.partial(
        pl.kernel,
        mesh=mesh,
        out_type=jax.ShapeDtypeStruct((B, D), jnp.float32),
        scratch_types=[
            pltpu.VMEM((_NBUF, C), jnp.int32),
            pltpu.VMEM((_NBUF, C, D), jnp.float32),
            pltpu.SemaphoreType.DMA((_NBUF,)),
            pltpu.SemaphoreType.DMA((_NBUF,)),
        ],
        compiler_params=pltpu.CompilerParams(use_tc_tiling_on_sc=False),
    )
    def k(table_hbm, idx_hbm, out_hbm, idx_v, rows_v, gsem, ssem):
        wid = lax.axis_index("s") * NC + lax.axis_index("c")
        base = wid * b_per_w

        def start_gather(i, p):
            pltpu.sync_copy(idx_hbm.at[pl.ds(base + i * C, C)], idx_v.at[p])
            pltpu.async_copy(table_hbm.at[idx_v.at[p]], rows_v.at[p], gsem.at[p])

        def wait_gather(p):
            pltpu.make_async_copy(
                table_hbm.at[idx_v.at[p]], rows_v.at[p], gsem.at[p]
            ).wait()

        def start_store(i, p):
            pltpu.async_copy(
                rows_v.at[p], out_hbm.at[pl.ds(base + i * C, C)], ssem.at[p]
            )

        def wait_store(i, p):
            pltpu.make_async_copy(
                rows_v.at[p], out_hbm.at[pl.ds(base + i * C, C)], ssem.at[p]
            ).wait()

        for p in range(_LEAD):
            start_gather(p, p)

        def body(j, carry):
            for p in range(_NBUF):
                i = j * _NBUF + p
                wait_gather(p)
                start_store(i, p)
                q = (p + _LEAD) % _NBUF

                @pl.when(i + _LEAD < n_chunks)
                def _issue():
                    @pl.when(i >= _LEAD)
                    def _drain():
                        wait_store(i - _LEAD, q)

                    start_gather(i + _LEAD, q)

            return carry

        lax.fori_loop(0, n_chunks // _NBUF, body, 0)
        for i in range(n_chunks - _LEAD, n_chunks):
            wait_store(i, i % _NBUF)

    return k


def kernel(x, table):
    BATCH, HIST = x.shape
    V, D = table.shape
    B = BATCH * HIST
    xf = x.reshape(B).astype(jnp.int32)
    out = _make_gather(V, D, B, 400)(table, xf)
    return out.reshape(BATCH, HIST, D)
